# baseline (speedup 1.0000x reference)
import functools
import math

import jax
import jax.numpy as jnp
from jax import lax
from jax.experimental import pallas as pl
from jax.experimental.pallas import tpu as pltpu
from jax.experimental.pallas import tpu_sc as plsc

_BATCH = 4096
_WINDOW = 200
_EMBED = 64
_B = _BATCH * _WINDOW
_NC, _NS = 2, 16
_NW = _NC * _NS
_BPW = _B // _NW
_CH = 800
_NCHUNK = _BPW // _CH
_SCALE = math.sqrt(_EMBED)
_LANES = 16
_GSIZES = (128, 128, 128, 128, 128, 128, 32)

_mesh = plsc.VectorSubcoreMesh(core_axis_name="c", subcore_axis_name="s")


@functools.partial(
    pl.kernel,
    out_type=jax.ShapeDtypeStruct((_B, _EMBED), jnp.float32),
    mesh=_mesh,
    scratch_types=[
        pltpu.VMEM((_CH,), jnp.int32),
        pltpu.VMEM((_CH, _EMBED), jnp.float32),
        pltpu.VMEM((_WINDOW, _EMBED), jnp.float32),
        pltpu.SemaphoreType.DMA,
    ],
    compiler_params=pltpu.CompilerParams(use_tc_tiling_on_sc=False),
)
def _emb_pe_kernel(x_hbm, table_hbm, pos_hbm, out_hbm, idx_v, rows_v, pos_v, sem):
    wid = lax.axis_index("s") * _NC + lax.axis_index("c")
    base = wid * _BPW
    pltpu.sync_copy(pos_hbm, pos_v)

    @pl.loop(0, _NCHUNK)
    def _chunk(g):
        cbase = base + g * _CH
        pltpu.sync_copy(x_hbm.at[pl.ds(cbase, _CH)], idx_v)
        copies = []
        off = 0
        for sz in _GSIZES:
            copies.append(
                pltpu.async_copy(
                    table_hbm.at[idx_v.at[pl.ds(off, sz)]],
                    rows_v.at[pl.ds(off, sz)],
                    sem,
                )
            )
            off += sz
        for c in copies:
            c.wait()

        @pl.loop(0, _WINDOW)
        def _row(j):
            for s in range(_EMBED // _LANES):
                sl = pl.ds(s * _LANES, _LANES)
                p = pos_v[j, sl]
                for w in range(_CH // _WINDOW):
                    r = w * _WINDOW
                    rows_v[r + j, sl] = rows_v[r + j, sl] * _SCALE + p

        pltpu.sync_copy(rows_v, out_hbm.at[pl.ds(cbase, _CH)])


def kernel(x, table, pos_encoding):
    out = _emb_pe_kernel(x.reshape(_B), table, pos_encoding)
    return out.reshape(_BATCH, _WINDOW, _EMBED)
